# baseline (speedup 1.0000x reference)
import functools

import jax
import jax.numpy as jnp
from jax import lax
from jax.experimental import pallas as pl
from jax.experimental.pallas import tpu as pltpu
from jax.experimental.pallas import tpu_sc as plsc

_M = 1024
_K = 128
_N = 100000
_TN = 2000
_G = _N // _TN

_I32_MAX = 2**31 - 1


def _nn_body(a_ref, b_ref, bsq_ref, asq_ref, md_ref, lb_ref):
    i = pl.program_id(0)
    dot = lax.dot_general(
        a_ref[...], b_ref[...],
        dimension_numbers=(((1,), (1,)), ((), ())),
        preferred_element_type=jnp.float32,
    )
    dis = (asq_ref[...] + bsq_ref[0]) - 2.0 * dot
    tmin = jnp.min(dis, axis=1, keepdims=True)
    ii = lax.broadcasted_iota(jnp.int32, dis.shape, 1)
    targ = jnp.min(jnp.where(dis == tmin, ii, _I32_MAX),
                   axis=1, keepdims=True) + i * _TN

    @pl.when(i == 0)
    def _init():
        md_ref[...] = tmin
        lb_ref[...] = targ

    @pl.when(i > 0)
    def _merge():
        old = md_ref[...]
        better = tmin < old
        md_ref[...] = jnp.where(better, tmin, old)
        lb_ref[...] = jnp.where(better, targ, lb_ref[...])


def _nn_tc(a, b, bsq, asq):
    return pl.pallas_call(
        _nn_body,
        grid=(_G,),
        in_specs=[
            pl.BlockSpec((_M, _K), lambda i: (0, 0)),
            pl.BlockSpec((_TN, _K), lambda i: (i, 0)),
            pl.BlockSpec((1, 1, _TN), lambda i: (i, 0, 0)),
            pl.BlockSpec((_M, 1), lambda i: (0, 0)),
        ],
        out_specs=[
            pl.BlockSpec((_M, 1), lambda i: (0, 0)),
            pl.BlockSpec((_M, 1), lambda i: (0, 0)),
        ],
        out_shape=[
            jax.ShapeDtypeStruct((_M, 1), jnp.float32),
            jax.ShapeDtypeStruct((_M, 1), jnp.int32),
        ],
    )(a, b, bsq, asq)


_NC = 2
_NS = 16
_NW = _NC * _NS
_BPW = _M // _NW
_L = 16


def _sc_body(md_hbm, lb_hbm, r_hbm, out_hbm, idx_v, md_v, rg_v, out_v, sem):
    wid = lax.axis_index("s") * _NC + lax.axis_index("c")
    base = wid * _BPW
    pltpu.sync_copy(lb_hbm.at[pl.ds(base, _BPW)], idx_v)
    pltpu.async_copy(r_hbm.at[idx_v], rg_v, sem).wait()
    pltpu.sync_copy(md_hbm.at[pl.ds(base, _BPW)], md_v)
    for j in range(_BPW // _L):
        sl = pl.ds(j * _L, _L)
        r = rg_v[sl]
        out_v[sl] = md_v[sl] - r * r
    pltpu.sync_copy(out_v, out_hbm.at[pl.ds(base, _BPW)])


@functools.cache
def _sc_score():
    return pl.kernel(
        _sc_body,
        out_type=jax.ShapeDtypeStruct((_M,), jnp.float32),
        mesh=plsc.VectorSubcoreMesh(core_axis_name="c", subcore_axis_name="s"),
        scratch_types=[
            pltpu.VMEM((_BPW,), jnp.int32),
            pltpu.VMEM((_BPW,), jnp.float32),
            pltpu.VMEM((_BPW,), jnp.float32),
            pltpu.VMEM((_BPW,), jnp.float32),
            pltpu.SemaphoreType.DMA,
        ],
    )


def kernel(a, b, R):
    asq = jnp.sum(a ** 2, axis=1)[:, None]
    bsq = jnp.sum(b ** 2, axis=1)
    md2, lb2 = _nn_tc(a, b, bsq.reshape(_G, 1, _TN), asq)
    md = md2[:, 0]
    lb = lb2[:, 0]
    scorek = _sc_score()(md, lb, R)
    return (scorek, lb, md)


# speedup vs baseline: 1.0109x; 1.0109x over previous
import functools

import jax
import jax.numpy as jnp
from jax import lax
from jax.experimental import pallas as pl
from jax.experimental.pallas import tpu as pltpu
from jax.experimental.pallas import tpu_sc as plsc

_M = 1024
_K = 128
_N = 100000
_TN = 2000
_G = _N // _TN

_I32_MAX = 2**31 - 1


def _nn_body(a_ref, b_ref, bsq_ref, asq_ref, md_ref, lb_ref, ii_ref):
    i = pl.program_id(0)

    @pl.when(i == 0)
    def _iota():
        ii_ref[...] = lax.broadcasted_iota(
            jnp.int32, (_M, _TN), 1).astype(jnp.float32)
    dot2 = lax.dot_general(
        a_ref[...], b_ref[...],
        dimension_numbers=(((1,), (1,)), ((), ())),
        preferred_element_type=jnp.float32,
    )
    dis = (asq_ref[...] + bsq_ref[0]) + dot2
    tmin = jnp.min(dis, axis=1, keepdims=True)
    targ_f = jnp.min(jnp.where(dis == tmin, ii_ref[...], jnp.float32(3.0e38)),
                     axis=1, keepdims=True)
    targ = targ_f.astype(jnp.int32) + i * _TN

    @pl.when(i == 0)
    def _init():
        md_ref[...] = tmin
        lb_ref[...] = targ

    @pl.when(i > 0)
    def _merge():
        old = md_ref[...]
        better = tmin < old
        md_ref[...] = jnp.where(better, tmin, old)
        lb_ref[...] = jnp.where(better, targ, lb_ref[...])


def _nn_tc(a, b, bsq, asq):
    return pl.pallas_call(
        _nn_body,
        grid=(_G,),
        in_specs=[
            pl.BlockSpec((_M, _K), lambda i: (0, 0)),
            pl.BlockSpec((_TN, _K), lambda i: (i, 0)),
            pl.BlockSpec((1, 1, _TN), lambda i: (i, 0, 0)),
            pl.BlockSpec((_M, 1), lambda i: (0, 0)),
        ],
        out_specs=[
            pl.BlockSpec((_M, 1), lambda i: (0, 0)),
            pl.BlockSpec((_M, 1), lambda i: (0, 0)),
        ],
        out_shape=[
            jax.ShapeDtypeStruct((_M, 1), jnp.float32),
            jax.ShapeDtypeStruct((_M, 1), jnp.int32),
        ],
        scratch_shapes=[pltpu.VMEM((_M, _TN), jnp.float32)],
    )(a, b, bsq, asq)


_NC = 2
_NS = 16
_NW = _NC * _NS
_BPW = _M // _NW
_L = 16


def _sc_body(md_hbm, lb_hbm, r_hbm, out_hbm, idx_v, md_v, rg_v, out_v, sem):
    wid = lax.axis_index("s") * _NC + lax.axis_index("c")
    base = wid * _BPW
    pltpu.sync_copy(lb_hbm.at[pl.ds(base, _BPW)], idx_v)
    pltpu.async_copy(r_hbm.at[idx_v], rg_v, sem).wait()
    pltpu.sync_copy(md_hbm.at[pl.ds(base, _BPW)], md_v)
    for j in range(_BPW // _L):
        sl = pl.ds(j * _L, _L)
        r = rg_v[sl]
        out_v[sl] = md_v[sl] - r * r
    pltpu.sync_copy(out_v, out_hbm.at[pl.ds(base, _BPW)])


@functools.cache
def _sc_score():
    return pl.kernel(
        _sc_body,
        out_type=jax.ShapeDtypeStruct((_M,), jnp.float32),
        mesh=plsc.VectorSubcoreMesh(core_axis_name="c", subcore_axis_name="s"),
        scratch_types=[
            pltpu.VMEM((_BPW,), jnp.int32),
            pltpu.VMEM((_BPW,), jnp.float32),
            pltpu.VMEM((_BPW,), jnp.float32),
            pltpu.VMEM((_BPW,), jnp.float32),
            pltpu.SemaphoreType.DMA,
        ],
    )


def kernel(a, b, R):
    asq = jnp.sum(a ** 2, axis=1)[:, None]
    bsq = jnp.sum(b ** 2, axis=1)
    md2, lb2 = _nn_tc(a * -2.0, b, bsq.reshape(_G, 1, _TN), asq)
    md = md2[:, 0]
    lb = lb2[:, 0]
    scorek = _sc_score()(md, lb, R)
    return (scorek, lb, md)
